# Initial kernel scaffold; baseline (speedup 1.0000x reference)
import jax
import jax.numpy as jnp
from jax.experimental import pallas as pl


def kernel(input, z, scale_table, shift_table):
    raise NotImplementedError("write your pallas kernel here")


# speedup vs baseline: 743.8276x; 743.8276x over previous
import functools

import jax
import jax.numpy as jnp
from jax import lax
from jax.experimental import pallas as pl
from jax.experimental.pallas import tpu as pltpu
from jax.experimental.pallas import tpu_sc as plsc

N = 4194304
VOCAB = 100
TBL = 128

NC, NS, L = 2, 16, 16
NW = NC * NS
PER_W = N // NW
CHUNK = 16384
NCHUNK = PER_W // CHUNK


def _scale_shift_body(inp_hbm, z_hbm, scale_hbm, shift_hbm, out_hbm,
                      scale_v, shift_v, z_v, x_v):
    wid = lax.axis_index("s") * NC + lax.axis_index("c")
    base = wid * PER_W

    pltpu.sync_copy(scale_hbm, scale_v)
    pltpu.sync_copy(shift_hbm, shift_v)

    def chunk_body(ci, carry):
        off = base + ci * CHUNK
        pltpu.sync_copy(z_hbm.at[pl.ds(off, CHUNK)], z_v)
        pltpu.sync_copy(inp_hbm.at[pl.ds(off, CHUNK)], x_v)

        def vec_body(i, c):
            s = pl.ds(i * L, L)
            idx = z_v[s]
            sc = plsc.load_gather(scale_v, [idx])
            sh = plsc.load_gather(shift_v, [idx])
            x_v[s] = x_v[s] * sc + sh
            return c

        lax.fori_loop(0, CHUNK // L, vec_body, 0)
        pltpu.sync_copy(x_v, out_hbm.at[pl.ds(off, CHUNK)])
        return carry

    lax.fori_loop(0, NCHUNK, chunk_body, 0)


@jax.jit
def kernel(input, z, scale_table, shift_table):
    inp_flat = input.reshape(N)
    z_i32 = z.astype(jnp.int32)
    scale_flat = jnp.zeros((TBL,), jnp.float32).at[:VOCAB].set(
        scale_table.reshape(VOCAB))
    shift_flat = jnp.zeros((TBL,), jnp.float32).at[:VOCAB].set(
        shift_table.reshape(VOCAB))

    mesh = plsc.VectorSubcoreMesh(core_axis_name="c", subcore_axis_name="s")
    run = functools.partial(
        pl.kernel,
        mesh=mesh,
        compiler_params=pltpu.CompilerParams(needs_layout_passes=False),
        out_type=jax.ShapeDtypeStruct((N,), jnp.float32),
        scratch_types=[
            pltpu.VMEM((TBL,), jnp.float32),
            pltpu.VMEM((TBL,), jnp.float32),
            pltpu.VMEM((CHUNK,), jnp.int32),
            pltpu.VMEM((CHUNK,), jnp.float32),
        ],
    )(_scale_shift_body)
    out_flat = run(inp_flat, z_i32, scale_flat, shift_flat)
    return out_flat.reshape(N, 1)


# speedup vs baseline: 1567.2420x; 2.1070x over previous
import functools

import jax
import jax.numpy as jnp
from jax import lax
from jax.experimental import pallas as pl
from jax.experimental.pallas import tpu as pltpu
from jax.experimental.pallas import tpu_sc as plsc

N = 4194304
VOCAB = 100
TBL = 128

NC, NS, L = 2, 16, 16
NW = NC * NS
PER_W = N // NW
CHUNK = 8192
NBUF = 3
NCHUNK = PER_W // CHUNK


def _scale_shift_body(inp_hbm, z_hbm, scale_hbm, shift_hbm, out_hbm,
                      scale_v, shift_v,
                      z0, z1, z2, x0, x1, x2, o0, o1, o2,
                      si0, si1, si2, so0, so1, so2):
    zb, xb, ob = (z0, z1, z2), (x0, x1, x2), (o0, o1, o2)
    sin, sout = (si0, si1, si2), (so0, so1, so2)

    wid = lax.axis_index("s") * NC + lax.axis_index("c")
    base = wid * PER_W

    pltpu.sync_copy(scale_hbm, scale_v)
    pltpu.sync_copy(shift_hbm, shift_v)

    def start_in(ci):
        b = ci % NBUF
        off = base + ci * CHUNK
        dz = pltpu.async_copy(z_hbm.at[pl.ds(off, CHUNK)], zb[b], sin[b])
        dx = pltpu.async_copy(inp_hbm.at[pl.ds(off, CHUNK)], xb[b], sin[b])
        return dz, dx

    indescs = {ci: start_in(ci) for ci in range(min(NBUF, NCHUNK))}
    outdescs = {}

    for ci in range(NCHUNK):
        b = ci % NBUF
        dz, dx = indescs.pop(ci)
        dz.wait()
        dx.wait()
        if ci >= NBUF:
            outdescs.pop(ci - NBUF).wait()

        z_v, x_v, o_v = zb[b], xb[b], ob[b]

        @plsc.parallel_loop(0, CHUNK // L, unroll=8)
        def _compute(i, z_v=z_v, x_v=x_v, o_v=o_v):
            s = pl.ds(i * L, L)
            idx = z_v[s]
            sc = plsc.load_gather(scale_v, [idx])
            sh = plsc.load_gather(shift_v, [idx])
            o_v[s] = x_v[s] * sc + sh

        if ci + NBUF < NCHUNK:
            indescs[ci + NBUF] = start_in(ci + NBUF)
        off = base + ci * CHUNK
        outdescs[ci] = pltpu.async_copy(o_v, out_hbm.at[pl.ds(off, CHUNK)],
                                        sout[b])

    for ci in sorted(outdescs):
        outdescs[ci].wait()


@jax.jit
def kernel(input, z, scale_table, shift_table):
    inp_flat = input.reshape(N)
    z_i32 = z.astype(jnp.int32)
    scale_flat = jnp.zeros((TBL,), jnp.float32).at[:VOCAB].set(
        scale_table.reshape(VOCAB))
    shift_flat = jnp.zeros((TBL,), jnp.float32).at[:VOCAB].set(
        shift_table.reshape(VOCAB))

    mesh = plsc.VectorSubcoreMesh(core_axis_name="c", subcore_axis_name="s")
    run = functools.partial(
        pl.kernel,
        mesh=mesh,
        compiler_params=pltpu.CompilerParams(needs_layout_passes=False),
        out_type=jax.ShapeDtypeStruct((N,), jnp.float32),
        scratch_types=[
            pltpu.VMEM((TBL,), jnp.float32),
            pltpu.VMEM((TBL,), jnp.float32),
            pltpu.VMEM((CHUNK,), jnp.int32),
            pltpu.VMEM((CHUNK,), jnp.int32),
            pltpu.VMEM((CHUNK,), jnp.int32),
            pltpu.VMEM((CHUNK,), jnp.float32),
            pltpu.VMEM((CHUNK,), jnp.float32),
            pltpu.VMEM((CHUNK,), jnp.float32),
            pltpu.VMEM((CHUNK,), jnp.float32),
            pltpu.VMEM((CHUNK,), jnp.float32),
            pltpu.VMEM((CHUNK,), jnp.float32),
            pltpu.SemaphoreType.DMA,
            pltpu.SemaphoreType.DMA,
            pltpu.SemaphoreType.DMA,
            pltpu.SemaphoreType.DMA,
            pltpu.SemaphoreType.DMA,
            pltpu.SemaphoreType.DMA,
        ],
    )(_scale_shift_body)
    out_flat = run(inp_flat, z_i32, scale_flat, shift_flat)
    return out_flat.reshape(N, 1)
